# baseline (speedup 1.0000x reference)
import functools

import jax
import jax.numpy as jnp
from jax import lax
from jax.experimental import pallas as pl
from jax.experimental.pallas import tpu as pltpu
from jax.experimental.pallas import tpu_sc as plsc

N = 10000
EDG = 320000
F = 128
R = 8

NC, NS = 2, 16
H = N // 2
K = 128
CH = 157
NBUF = 3
EPT = K * CH
ETOT = NS * EPT
NACC = 5008
RPT = 320
LAST_Z = NACC - RPT * (NS - 1)
LAST_C = H - RPT * (NS - 1)
PADTGT = 1 << 29

BN = 1000
BF = 1000


def _mm_body(e_ref, w_ref, o_ref):
    o_ref[...] = jnp.dot(e_ref[...], w_ref[...], preferred_element_type=jnp.float32)


def _finish_body(p_ref, o_ref):
    o_ref[...] = jnp.maximum(p_ref[0], 0.0)


_mesh = plsc.VectorSubcoreMesh(
    core_axis_name="c", subcore_axis_name="s", num_cores=NC, num_subcores=NS
)


@functools.partial(
    pl.kernel,
    out_type=jax.ShapeDtypeStruct((NC, H, F), jnp.float32),
    mesh=_mesh,
    compiler_params=pltpu.CompilerParams(needs_layout_passes=False),
    scratch_types=[
        pltpu.VMEM((CH, K), jnp.int32),
        pltpu.VMEM((CH, K), jnp.int32),
        pltpu.VMEM((16,), jnp.int32),
        *([pltpu.VMEM((K, F), jnp.float32)] * NBUF),
        pltpu.VMEM_SHARED((NACC, F), jnp.float32),
        *([pltpu.SemaphoreType.DMA] * NBUF),
    ],
)
def _sc_scatter(t_hbm, fidx_hbm, tgt_hbm, zrows_hbm, out_hbm,
                fidx_v, tgt_v, cnt_v, *rest):
    bufs = rest[:NBUF]
    accum = rest[NBUF]
    gsem = rest[NBUF + 1:]
    ssem = gsem
    c = lax.axis_index("c")
    s = lax.axis_index("s")

    @pl.when(s < NS - 1)
    def _():
        pltpu.sync_copy(zrows_hbm, accum.at[pl.ds(s * RPT, RPT)])
    @pl.when(s == NS - 1)
    def _():
        pltpu.sync_copy(zrows_hbm.at[pl.ds(0, LAST_Z)],
                        accum.at[pl.ds((NS - 1) * RPT, LAST_Z)])
    plsc.subcore_barrier()

    pltpu.sync_copy(fidx_hbm.at[s], fidx_v)
    pltpu.sync_copy(tgt_hbm.at[s], tgt_v)

    lo = c * H
    lanes = jnp.arange(16, dtype=jnp.int32)
    zero16 = jnp.zeros((16,), jnp.int32)

    def comp_body(i, offv):
        row = i // (K // 16)
        col = (i % (K // 16)) * 16
        t = tgt_v[row, pl.ds(col, 16)]
        f = fidx_v[row, pl.ds(col, 16)]
        tl = t - lo
        own = (tl >= 0) & (tl < H)
        cnt = plsc.all_reduce_population_count(own)
        sel = lanes < cnt
        pos = offv + lanes
        prow = pos >> 7
        pcol = pos & (K - 1)
        plsc.store_compressed(cnt_v.at[...], f, mask=own)
        fc = cnt_v[...]
        plsc.store_scatter(fidx_v, [prow, pcol], fc, mask=sel)
        plsc.store_compressed(cnt_v.at[...], tl, mask=own)
        tc = cnt_v[...]
        plsc.store_scatter(tgt_v, [prow, pcol], tc, mask=sel)
        return offv + cnt

    offv = lax.fori_loop(0, EPT // 16, comp_body, zero16)
    cnt_v[...] = offv
    off = cnt_v[...][0]

    nch = (off + K - 1) // K
    end = nch * K
    zeros16 = jnp.zeros((16,), jnp.int32)
    dumps16 = jnp.full((16,), H, jnp.int32)
    for b in range(K // 16):
        pos = off + b * 16 + lanes
        m = pos < end
        prow = pos >> 7
        pcol = pos & (K - 1)
        plsc.store_scatter(fidx_v, [prow, pcol], zeros16, mask=m)
        plsc.store_scatter(tgt_v, [prow, pcol], dumps16, mask=m)

    def gather(ch, j):
        pltpu.async_copy(t_hbm.at[fidx_v.at[ch]], bufs[j], gsem[j])

    def wait_gather(ch, j):
        pltpu.make_async_copy(t_hbm.at[fidx_v.at[ch]], bufs[j], gsem[j]).wait()

    def scatter(ch, j):
        pltpu.async_copy(bufs[j], accum.at[tgt_v.at[ch]], ssem[j], add=True)

    def wait_scatter(ch, j):
        pltpu.make_async_copy(bufs[j], accum.at[tgt_v.at[ch]], ssem[j]).wait()

    def pipe_step(ch, j):
        @pl.when(ch < nch)
        def _():
            wait_gather(ch, j)
            scatter(ch, j)
        d = ch - (NBUF - 2)
        if not (isinstance(d, int) and d < 0):
            @pl.when(d < nch)
            def _():
                wait_scatter(d, (j + 2) % NBUF)
        g = ch + 2
        @pl.when(g < nch)
        def _():
            gather(g, (j + 2) % NBUF)

    for ch in range(2):
        @pl.when(ch < nch)
        def _():
            gather(ch, ch % NBUF)
    for ch in range(2):
        pipe_step(ch, ch % NBUF)

    def body(i, carry):
        base = NBUF * i + 2
        for jj in range(NBUF):
            pipe_step(base + jj, (2 + jj) % NBUF)
        return carry

    lax.fori_loop(0, (CH - 4) // NBUF, body, 0)

    for ch in range(CH - 2, CH + 1):
        pipe_step(ch, ch % NBUF)

    plsc.subcore_barrier()
    @pl.when(s < NS - 1)
    def _():
        pltpu.sync_copy(accum.at[pl.ds(s * RPT, RPT)],
                        out_hbm.at[c, pl.ds(s * RPT, RPT)])
    @pl.when(s == NS - 1)
    def _():
        pltpu.sync_copy(accum.at[pl.ds((NS - 1) * RPT, LAST_C)],
                        out_hbm.at[c, pl.ds((NS - 1) * RPT, LAST_C)])


def kernel(entity_embeddings, weight, edge_index, edge_type):
    src = edge_index[0]
    tgt = edge_index[1]
    flat_idx = src * R + edge_type

    pad = ETOT - EDG
    fidx = jnp.concatenate(
        [flat_idx, jnp.zeros((pad,), jnp.int32)]).reshape(NS, CH, K)
    tgtp = jnp.concatenate(
        [tgt, jnp.full((pad,), PADTGT, jnp.int32)]).reshape(NS, CH, K)
    zrows = jnp.zeros((RPT, F), jnp.float32)

    w_cat = jnp.transpose(weight, (1, 0, 2)).reshape(F, R * F)
    t = pl.pallas_call(
        _mm_body,
        grid=(N // BN,),
        in_specs=[
            pl.BlockSpec((BN, F), lambda i: (i, 0)),
            pl.BlockSpec((F, R * F), lambda i: (0, 0)),
        ],
        out_specs=pl.BlockSpec((BN, R * F), lambda i: (i, 0)),
        out_shape=jax.ShapeDtypeStruct((N, R * F), jnp.float32),
    )(entity_embeddings, w_cat)
    t_flat = t.reshape(R * N, F)

    partials = _sc_scatter(t_flat, fidx, tgtp, zrows)

    out = pl.pallas_call(
        _finish_body,
        grid=(N // BF,),
        in_specs=[pl.BlockSpec((1, BF, F),
                               lambda i: (i // (H // BF), i % (H // BF), 0))],
        out_specs=pl.BlockSpec((BF, F), lambda i: (i, 0)),
        out_shape=jax.ShapeDtypeStruct((N, F), jnp.float32),
    )(partials)
    return out


# speedup vs baseline: 1.0237x; 1.0237x over previous
import functools

import jax
import jax.numpy as jnp
from jax import lax
from jax.experimental import pallas as pl
from jax.experimental.pallas import tpu as pltpu
from jax.experimental.pallas import tpu_sc as plsc

N = 10000
EDG = 320000
F = 128
R = 8

NC, NS = 2, 16
H = N // 2
K = 128
CH = 157
NBUF = 3
EPT = K * CH
ETOT = NS * EPT
NACC = 5008
RPT = 320
LAST_Z = NACC - RPT * (NS - 1)
LAST_C = H - RPT * (NS - 1)
PADTGT = (1 << 14) - 1

BN = 1000
BF = 1000


def _mm_body(e_ref, w_ref, o_ref):
    o_ref[0] = jnp.dot(e_ref[...], w_ref[0], preferred_element_type=jnp.float32)


def _finish_body(p_ref, o_ref):
    o_ref[...] = jnp.maximum(p_ref[0], 0.0)


_mesh = plsc.VectorSubcoreMesh(
    core_axis_name="c", subcore_axis_name="s", num_cores=NC, num_subcores=NS
)


@functools.partial(
    pl.kernel,
    out_type=jax.ShapeDtypeStruct((NC, H, F), jnp.float32),
    mesh=_mesh,
    compiler_params=pltpu.CompilerParams(needs_layout_passes=False),
    scratch_types=[
        pltpu.VMEM((CH, K), jnp.int32),
        pltpu.VMEM((CH, K), jnp.int32),
        pltpu.VMEM((16,), jnp.int32),
        *([pltpu.VMEM((K, F), jnp.float32)] * NBUF),
        pltpu.VMEM_SHARED((NACC, F), jnp.float32),
        *([pltpu.SemaphoreType.DMA] * NBUF),
    ],
)
def _sc_scatter(t_hbm, praw_hbm, zrows_hbm, out_hbm,
                fidx_v, tgt_v, cnt_v, *rest):
    bufs = rest[:NBUF]
    accum = rest[NBUF]
    gsem = rest[NBUF + 1:]
    ssem = gsem
    c = lax.axis_index("c")
    s = lax.axis_index("s")

    @pl.when(s < NS - 1)
    def _():
        pltpu.sync_copy(zrows_hbm, accum.at[pl.ds(s * RPT, RPT)])
    @pl.when(s == NS - 1)
    def _():
        pltpu.sync_copy(zrows_hbm.at[pl.ds(0, LAST_Z)],
                        accum.at[pl.ds((NS - 1) * RPT, LAST_Z)])
    plsc.subcore_barrier()

    pltpu.sync_copy(praw_hbm.at[s], fidx_v)

    lo = c * H
    lanes = jnp.arange(16, dtype=jnp.int32)
    zero16 = jnp.zeros((16,), jnp.int32)

    lobase = lo << 17

    def comp_body(i, offv):
        row = i // (K // 16)
        col = (i % (K // 16)) * 16
        p = fidx_v[row, pl.ds(col, 16)]
        tl = (p >> 17) - lo
        own = (tl >= 0) & (tl < H)
        cnt = plsc.all_reduce_population_count(own)
        sel = lanes < cnt
        pos = offv + lanes
        prow = pos >> 7
        pcol = pos & (K - 1)
        plsc.store_compressed(cnt_v.at[...], p - lobase, mask=own)
        pc = cnt_v[...]
        plsc.store_scatter(fidx_v, [prow, pcol], pc & ((1 << 17) - 1),
                           mask=sel)
        plsc.store_scatter(tgt_v, [prow, pcol], pc >> 17, mask=sel)
        return offv + cnt

    offv = lax.fori_loop(0, EPT // 16, comp_body, zero16)
    cnt_v[...] = offv
    off = cnt_v[...][0]

    nch = (off + K - 1) // K
    end = nch * K
    zeros16 = jnp.zeros((16,), jnp.int32)
    dumps16 = jnp.full((16,), H, jnp.int32)
    for b in range(K // 16):
        pos = off + b * 16 + lanes
        m = pos < end
        prow = pos >> 7
        pcol = pos & (K - 1)
        plsc.store_scatter(fidx_v, [prow, pcol], zeros16, mask=m)
        plsc.store_scatter(tgt_v, [prow, pcol], dumps16, mask=m)

    def gather(ch, j):
        pltpu.async_copy(t_hbm.at[fidx_v.at[ch]], bufs[j], gsem[j])

    def wait_gather(ch, j):
        pltpu.make_async_copy(t_hbm.at[fidx_v.at[ch]], bufs[j], gsem[j]).wait()

    def scatter(ch, j):
        pltpu.async_copy(bufs[j], accum.at[tgt_v.at[ch]], ssem[j], add=True)

    def wait_scatter(ch, j):
        pltpu.make_async_copy(bufs[j], accum.at[tgt_v.at[ch]], ssem[j]).wait()

    def pipe_step(ch, j):
        @pl.when(ch < nch)
        def _():
            wait_gather(ch, j)
            scatter(ch, j)
        d = ch - (NBUF - 2)
        if not (isinstance(d, int) and d < 0):
            @pl.when(d < nch)
            def _():
                wait_scatter(d, (j + 2) % NBUF)
        g = ch + 2
        @pl.when(g < nch)
        def _():
            gather(g, (j + 2) % NBUF)

    for ch in range(2):
        @pl.when(ch < nch)
        def _():
            gather(ch, ch % NBUF)
    for ch in range(2):
        pipe_step(ch, ch % NBUF)

    def body(i, carry):
        base = NBUF * i + 2
        for jj in range(NBUF):
            pipe_step(base + jj, (2 + jj) % NBUF)
        return carry

    lax.fori_loop(0, (CH - 4) // NBUF, body, 0)

    for ch in range(CH - 2, CH + 1):
        pipe_step(ch, ch % NBUF)

    plsc.subcore_barrier()
    @pl.when(s < NS - 1)
    def _():
        pltpu.sync_copy(accum.at[pl.ds(s * RPT, RPT)],
                        out_hbm.at[c, pl.ds(s * RPT, RPT)])
    @pl.when(s == NS - 1)
    def _():
        pltpu.sync_copy(accum.at[pl.ds((NS - 1) * RPT, LAST_C)],
                        out_hbm.at[c, pl.ds((NS - 1) * RPT, LAST_C)])


def kernel(entity_embeddings, weight, edge_index, edge_type):
    src = edge_index[0]
    tgt = edge_index[1]
    flat_idx = edge_type * N + src

    pad = ETOT - EDG
    packed = (tgt << 17) | flat_idx
    praw = jnp.concatenate(
        [packed, jnp.full((pad,), (PADTGT << 17), jnp.int32)]
    ).reshape(NS, CH, K)
    zrows = jnp.zeros((RPT, F), jnp.float32)

    t = pl.pallas_call(
        _mm_body,
        grid=(N // BN, R),
        in_specs=[
            pl.BlockSpec((BN, F), lambda i, r: (i, 0)),
            pl.BlockSpec((1, F, F), lambda i, r: (r, 0, 0)),
        ],
        out_specs=pl.BlockSpec((1, BN, F), lambda i, r: (r, i, 0)),
        out_shape=jax.ShapeDtypeStruct((R, N, F), jnp.float32),
    )(entity_embeddings, weight)
    t_flat = t.reshape(R * N, F)

    partials = _sc_scatter(t_flat, praw, zrows)

    out = pl.pallas_call(
        _finish_body,
        grid=(N // BF,),
        in_specs=[pl.BlockSpec((1, BF, F),
                               lambda i: (i // (H // BF), i % (H // BF), 0))],
        out_specs=pl.BlockSpec((BF, F), lambda i: (i, 0)),
        out_shape=jax.ShapeDtypeStruct((N, F), jnp.float32),
    )(partials)
    return out


# speedup vs baseline: 1.0304x; 1.0066x over previous
import functools

import jax
import jax.numpy as jnp
from jax import lax
from jax.experimental import pallas as pl
from jax.experimental.pallas import tpu as pltpu
from jax.experimental.pallas import tpu_sc as plsc

N = 10000
EDG = 320000
F = 128
R = 8

NC, NS = 2, 16
H = N // 2
K = 128
CH = 157
NBUF = 3
EPT = K * CH
ETOT = NS * EPT
NACC = 5008
RPT = 320
LAST_Z = NACC - RPT * (NS - 1)
LAST_C = H - RPT * (NS - 1)
PADTGT = (1 << 14) - 1

BN = 1000
BF = 1000


def _mm_body(e_ref, w_ref, o_ref):
    o_ref[0] = jnp.dot(e_ref[...], w_ref[0], preferred_element_type=jnp.float32)


def _finish_body(p_ref, o_ref):
    o_ref[...] = jnp.maximum(p_ref[0], 0.0)


_mesh = plsc.VectorSubcoreMesh(
    core_axis_name="c", subcore_axis_name="s", num_cores=NC, num_subcores=NS
)


@functools.partial(
    pl.kernel,
    out_type=jax.ShapeDtypeStruct((NC, H, F), jnp.float32),
    mesh=_mesh,
    compiler_params=pltpu.CompilerParams(needs_layout_passes=False),
    scratch_types=[
        pltpu.VMEM((CH, K), jnp.int32),
        pltpu.VMEM((CH, K), jnp.int32),
        pltpu.VMEM((16,), jnp.int32),
        *([pltpu.VMEM((K, F), jnp.float32)] * NBUF),
        pltpu.VMEM_SHARED((NACC, F), jnp.float32),
        *([pltpu.SemaphoreType.DMA] * (NBUF + 1)),
    ],
)
def _sc_scatter(t_hbm, praw_hbm, zrows_hbm, out_hbm,
                fidx_v, tgt_v, cnt_v, *rest):
    bufs = rest[:NBUF]
    accum = rest[NBUF]
    gsem = rest[NBUF + 1:NBUF + 1 + NBUF]
    ssem = gsem
    c = lax.axis_index("c")
    s = lax.axis_index("s")

    @pl.when(s < NS - 1)
    def _():
        pltpu.async_copy(zrows_hbm, accum.at[pl.ds(s * RPT, RPT)], rest[-1])
    @pl.when(s == NS - 1)
    def _():
        pltpu.async_copy(zrows_hbm.at[pl.ds(0, LAST_Z)],
                         accum.at[pl.ds((NS - 1) * RPT, LAST_Z)], rest[-1])

    pltpu.sync_copy(praw_hbm.at[s], fidx_v)

    lo = c * H
    lanes = jnp.arange(16, dtype=jnp.int32)
    zero16 = jnp.zeros((16,), jnp.int32)

    lobase = lo << 17

    def comp_body(i, offv):
        row = i // (K // 16)
        col = (i % (K // 16)) * 16
        p = fidx_v[row, pl.ds(col, 16)]
        tl = (p >> 17) - lo
        own = (tl >= 0) & (tl < H)
        cnt = plsc.all_reduce_population_count(own)
        sel = lanes < cnt
        pos = offv + lanes
        prow = pos >> 7
        pcol = pos & (K - 1)
        plsc.store_compressed(cnt_v.at[...], p - lobase, mask=own)
        pc = cnt_v[...]
        plsc.store_scatter(fidx_v, [prow, pcol], pc & ((1 << 17) - 1),
                           mask=sel)
        plsc.store_scatter(tgt_v, [prow, pcol], pc >> 17, mask=sel)
        return offv + cnt

    offv = lax.fori_loop(0, EPT // 16, comp_body, zero16)
    cnt_v[...] = offv
    off = cnt_v[...][0]

    nch = (off + K - 1) // K
    end = nch * K
    zeros16 = jnp.zeros((16,), jnp.int32)
    dumps16 = jnp.full((16,), H, jnp.int32)
    for b in range(K // 16):
        pos = off + b * 16 + lanes
        m = pos < end
        prow = pos >> 7
        pcol = pos & (K - 1)
        plsc.store_scatter(fidx_v, [prow, pcol], zeros16, mask=m)
        plsc.store_scatter(tgt_v, [prow, pcol], dumps16, mask=m)

    @pl.when(s < NS - 1)
    def _():
        pltpu.make_async_copy(
            zrows_hbm, accum.at[pl.ds(s * RPT, RPT)], rest[-1]).wait()
    @pl.when(s == NS - 1)
    def _():
        pltpu.make_async_copy(
            zrows_hbm.at[pl.ds(0, LAST_Z)],
            accum.at[pl.ds((NS - 1) * RPT, LAST_Z)], rest[-1]).wait()
    plsc.subcore_barrier()

    def gather(ch, j):
        pltpu.async_copy(t_hbm.at[fidx_v.at[ch]], bufs[j], gsem[j])

    def wait_gather(ch, j):
        pltpu.make_async_copy(t_hbm.at[fidx_v.at[ch]], bufs[j], gsem[j]).wait()

    def scatter(ch, j):
        pltpu.async_copy(bufs[j], accum.at[tgt_v.at[ch]], ssem[j], add=True)

    def wait_scatter(ch, j):
        pltpu.make_async_copy(bufs[j], accum.at[tgt_v.at[ch]], ssem[j]).wait()

    def pipe_step(ch, j):
        @pl.when(ch < nch)
        def _():
            wait_gather(ch, j)
            scatter(ch, j)
        d = ch - (NBUF - 2)
        if not (isinstance(d, int) and d < 0):
            @pl.when(d < nch)
            def _():
                wait_scatter(d, (j + 2) % NBUF)
        g = ch + 2
        @pl.when(g < nch)
        def _():
            gather(g, (j + 2) % NBUF)

    for ch in range(2):
        @pl.when(ch < nch)
        def _():
            gather(ch, ch % NBUF)
    for ch in range(2):
        pipe_step(ch, ch % NBUF)

    def body(i, carry):
        base = NBUF * i + 2
        for jj in range(NBUF):
            pipe_step(base + jj, (2 + jj) % NBUF)
        return carry

    lax.fori_loop(0, (CH - 4) // NBUF, body, 0)

    for ch in range(CH - 2, CH + 1):
        pipe_step(ch, ch % NBUF)

    plsc.subcore_barrier()
    @pl.when(s < NS - 1)
    def _():
        pltpu.sync_copy(accum.at[pl.ds(s * RPT, RPT)],
                        out_hbm.at[c, pl.ds(s * RPT, RPT)])
    @pl.when(s == NS - 1)
    def _():
        pltpu.sync_copy(accum.at[pl.ds((NS - 1) * RPT, LAST_C)],
                        out_hbm.at[c, pl.ds((NS - 1) * RPT, LAST_C)])


def kernel(entity_embeddings, weight, edge_index, edge_type):
    src = edge_index[0]
    tgt = edge_index[1]
    flat_idx = edge_type * N + src

    pad = ETOT - EDG
    packed = (tgt << 17) | flat_idx
    praw = jnp.concatenate(
        [packed, jnp.full((pad,), (PADTGT << 17), jnp.int32)]
    ).reshape(NS, CH, K)
    zrows = jnp.zeros((RPT, F), jnp.float32)

    t = pl.pallas_call(
        _mm_body,
        grid=(N // BN, R),
        in_specs=[
            pl.BlockSpec((BN, F), lambda i, r: (i, 0)),
            pl.BlockSpec((1, F, F), lambda i, r: (r, 0, 0)),
        ],
        out_specs=pl.BlockSpec((1, BN, F), lambda i, r: (r, i, 0)),
        out_shape=jax.ShapeDtypeStruct((R, N, F), jnp.float32),
    )(entity_embeddings, weight)
    t_flat = t.reshape(R * N, F)

    partials = _sc_scatter(t_flat, praw, zrows)

    out = pl.pallas_call(
        _finish_body,
        grid=(N // BF,),
        in_specs=[pl.BlockSpec((1, BF, F),
                               lambda i: (i // (H // BF), i % (H // BF), 0))],
        out_specs=pl.BlockSpec((BF, F), lambda i: (i, 0)),
        out_shape=jax.ShapeDtypeStruct((N, F), jnp.float32),
    )(partials)
    return out
